# baseline (speedup 1.0000x reference)
import functools

import jax
import jax.numpy as jnp
from jax import lax
from jax.experimental import pallas as pl
from jax.experimental.pallas import tpu as pltpu
from jax.experimental.pallas import tpu_sc as plsc

B = 16384
D = 128

_info = plsc.get_sparse_core_info()
_NC, _NS = _info.num_cores, _info.num_subcores
_NW = _NC * _NS
_BPW = B // _NW
_CH = 128
_NCH = _BPW // _CH

_mesh = plsc.VectorSubcoreMesh(core_axis_name="c", subcore_axis_name="s")


@functools.partial(
    pl.kernel,
    out_type=jax.ShapeDtypeStruct((B, 2, D), jnp.float32),
    mesh=_mesh,
    scratch_types=[
        pltpu.VMEM((_NCH, _CH), jnp.int32),
        pltpu.VMEM((_NCH, _CH, D), jnp.float32),
        pltpu.VMEM((_CH, D), jnp.float32),
        pltpu.SemaphoreType.DMA,
    ],
)
def _first_layer_sc(loc_hbm, x_hbm, table_hbm, out_hbm, idx_v, emb_v, x_v, sem):
    wid = lax.axis_index("s") * _NC + lax.axis_index("c")
    base = wid * _BPW

    for j in range(_NCH):
        pltpu.sync_copy(loc_hbm.at[pl.ds(base + j * _CH, _CH)], idx_v.at[j])

    gathers = [
        pltpu.async_copy(table_hbm.at[idx_v.at[j]], emb_v.at[j], sem)
        for j in range(_NCH)
    ]

    for j in range(_NCH):
        pltpu.sync_copy(x_hbm.at[pl.ds(base + j * _CH, _CH)], x_v)
        pltpu.sync_copy(x_v, out_hbm.at[pl.ds(base + j * _CH, _CH), 1])

    for g in gathers:
        g.wait()
    for j in range(_NCH):
        pltpu.sync_copy(emb_v.at[j], out_hbm.at[pl.ds(base + j * _CH, _CH), 0])


def kernel(loc, x, embedding_table):
    out3 = _first_layer_sc(loc.astype(jnp.int32), x, embedding_table)
    return out3.reshape(B, 2 * D)
